# baseline (speedup 1.0000x reference)
import functools

import jax
import jax.numpy as jnp
from jax import lax
from jax.experimental import pallas as pl
from jax.experimental.pallas import tpu as pltpu
from jax.experimental.pallas import tpu_sc as plsc

_NC = 1000
_NW = 32
_NBUF = 4


def kernel(x):
    n0, n1 = x.shape
    rows_per_w = n0 // _NW
    xf = jnp.reshape(x, (-1,))
    mesh = plsc.VectorSubcoreMesh(core_axis_name="c", subcore_axis_name="s")

    @functools.partial(
        pl.kernel,
        out_type=jax.ShapeDtypeStruct((n0, n1, _NC), jnp.float32),
        mesh=mesh,
        scratch_types=[
            pltpu.VMEM((rows_per_w * n1,), jnp.int32),
            pltpu.VMEM((_NBUF, n1, _NC), jnp.float32),
            pltpu.SemaphoreType.DMA((_NBUF,)),
        ],
        compiler_params=pltpu.CompilerParams(
            use_tc_tiling_on_sc=False, needs_layout_passes=False),
    )
    def _sc(x_hbm, o_hbm, xv, buf, sem):
        wid = lax.axis_index("s") * 2 + lax.axis_index("c")
        r0 = wid * rows_per_w
        pltpu.sync_copy(x_hbm.at[pl.ds(r0 * n1, rows_per_w * n1)], xv)

        it16 = lax.iota(jnp.int32, 16)
        s0 = it16
        s1 = it16 + (n1 - 16)
        ones = jnp.full((16,), 1.0, jnp.float32)
        zeros = jnp.zeros((16,), jnp.float32)
        tail_mask = it16 < jnp.full((16,), _NC - 62 * 16, jnp.int32)

        def zrow_body(k, carry):
            s = k // 63
            t = k % 63
            plsc.store_scatter(
                buf.at[carry], [jnp.full((16,), s, jnp.int32), t * 16 + it16],
                zeros, mask=tail_mask | (t < jnp.full((16,), 62, jnp.int32)))
            return carry

        for b in range(_NBUF):
            lax.fori_loop(0, n1 * 63, zrow_body, b)

        def put(j, b):
            c0 = xv[pl.ds(j * n1, 16)]
            c1 = xv[pl.ds(j * n1 + n1 - 16, 16)]
            plsc.store_scatter(buf.at[b], [s0, c0], ones)
            plsc.store_scatter(buf.at[b], [s1, c1], ones)
            pltpu.make_async_copy(buf.at[b], o_hbm.at[r0 + j], sem.at[b]).start()

        def reclaim(j, b):
            pltpu.make_async_copy(buf.at[b], o_hbm.at[r0 + j], sem.at[b]).wait()
            c0 = xv[pl.ds(j * n1, 16)]
            c1 = xv[pl.ds(j * n1 + n1 - 16, 16)]
            plsc.store_scatter(buf.at[b], [s0, c0], zeros)
            plsc.store_scatter(buf.at[b], [s1, c1], zeros)

        for b in range(_NBUF):
            put(b, b)

        def body(g, carry):
            for b in range(_NBUF):
                j = g * _NBUF + b
                reclaim(j - _NBUF, b)
                put(j, b)
            return carry

        lax.fori_loop(1, rows_per_w // _NBUF, body, 0)

        for b in range(_NBUF):
            j = rows_per_w - _NBUF + b
            pltpu.make_async_copy(buf.at[b], o_hbm.at[r0 + j], sem.at[b]).wait()

    return _sc(xf)


# speedup vs baseline: 1.9766x; 1.9766x over previous
import functools

import jax
import jax.numpy as jnp
from jax import lax
from jax.experimental import pallas as pl
from jax.experimental.pallas import tpu as pltpu
from jax.experimental.pallas import tpu_sc as plsc

_NC = 1000
_NW = 32
_NBUF = 2


def kernel(x):
    n0, n1 = x.shape
    rows_per_w = n0 // _NW
    xf = jnp.reshape(x, (-1,))
    mesh = plsc.VectorSubcoreMesh(core_axis_name="c", subcore_axis_name="s")

    @functools.partial(
        pl.kernel,
        out_type=jax.ShapeDtypeStruct((n0, n1, _NC), jnp.float32),
        mesh=mesh,
        scratch_types=[
            pltpu.VMEM((rows_per_w * n1,), jnp.int32),
            pltpu.VMEM((_NBUF, n1, _NC), jnp.float32),
            pltpu.SemaphoreType.DMA((_NBUF,)),
        ],
        compiler_params=pltpu.CompilerParams(
            use_tc_tiling_on_sc=True, needs_layout_passes=False),
    )
    def _sc(x_hbm, o_hbm, xv, buf, sem):
        wid = lax.axis_index("s") * 2 + lax.axis_index("c")
        r0 = wid * rows_per_w
        pltpu.sync_copy(x_hbm.at[pl.ds(r0 * n1, rows_per_w * n1)], xv)

        it16 = lax.iota(jnp.int32, 16)
        s0 = it16
        s1 = it16 + (n1 - 16)
        ones = jnp.full((16,), 1.0, jnp.float32)
        zeros = jnp.zeros((16,), jnp.float32)
        tail_mask = it16 < jnp.full((16,), _NC - 62 * 16, jnp.int32)

        def zrow_body(k, carry):
            s = k // 63
            t = k % 63
            plsc.store_scatter(
                buf.at[carry], [jnp.full((16,), s, jnp.int32), t * 16 + it16],
                zeros, mask=tail_mask | (t < jnp.full((16,), 62, jnp.int32)))
            return carry

        for b in range(_NBUF):
            lax.fori_loop(0, n1 * 63, zrow_body, b)

        def put(j, b):
            c0 = xv[pl.ds(j * n1, 16)]
            c1 = xv[pl.ds(j * n1 + n1 - 16, 16)]
            plsc.store_scatter(buf.at[b], [s0, c0], ones)
            plsc.store_scatter(buf.at[b], [s1, c1], ones)
            pltpu.make_async_copy(buf.at[b], o_hbm.at[r0 + j], sem.at[b]).start()

        def reclaim(j, b):
            pltpu.make_async_copy(buf.at[b], o_hbm.at[r0 + j], sem.at[b]).wait()
            c0 = xv[pl.ds(j * n1, 16)]
            c1 = xv[pl.ds(j * n1 + n1 - 16, 16)]
            plsc.store_scatter(buf.at[b], [s0, c0], zeros)
            plsc.store_scatter(buf.at[b], [s1, c1], zeros)

        for b in range(_NBUF):
            put(b, b)

        def body(g, carry):
            for b in range(_NBUF):
                j = g * _NBUF + b
                reclaim(j - _NBUF, b)
                put(j, b)
            return carry

        lax.fori_loop(1, rows_per_w // _NBUF, body, 0)

        for b in range(_NBUF):
            j = rows_per_w - _NBUF + b
            pltpu.make_async_copy(buf.at[b], o_hbm.at[r0 + j], sem.at[b]).wait()

    return _sc(xf)


# speedup vs baseline: 1.9789x; 1.0012x over previous
import functools

import jax
import jax.numpy as jnp
from jax import lax
from jax.experimental import pallas as pl
from jax.experimental.pallas import tpu as pltpu
from jax.experimental.pallas import tpu_sc as plsc

_NC = 1000
_NW = 32
_NBUF = 2


def kernel(x):
    n0, n1 = x.shape
    rows_per_w = n0 // _NW
    mesh = plsc.VectorSubcoreMesh(core_axis_name="c", subcore_axis_name="s")

    @functools.partial(
        pl.kernel,
        out_type=jax.ShapeDtypeStruct((n0, n1, _NC), jnp.float32),
        mesh=mesh,
        scratch_types=[
            pltpu.VMEM((rows_per_w, n1), jnp.int32),
            pltpu.VMEM((_NBUF, n1, _NC), jnp.float32),
            pltpu.SemaphoreType.DMA((_NBUF,)),
        ],
        compiler_params=pltpu.CompilerParams(
            use_tc_tiling_on_sc=True, needs_layout_passes=False),
    )
    def _sc(x_hbm, o_hbm, xv, buf, sem):
        wid = lax.axis_index("s") * 2 + lax.axis_index("c")
        r0 = wid * rows_per_w
        pltpu.sync_copy(x_hbm.at[pl.ds(r0, rows_per_w), :], xv)

        it16 = lax.iota(jnp.int32, 16)
        s0 = it16
        s1 = it16 + (n1 - 16)
        ones = jnp.full((16,), 1.0, jnp.float32)
        zeros = jnp.zeros((16,), jnp.float32)
        tail_mask = it16 < jnp.full((16,), _NC - 62 * 16, jnp.int32)

        def zrow_body(k, carry):
            s = k // 63
            t = k % 63
            plsc.store_scatter(
                buf.at[carry], [jnp.full((16,), s, jnp.int32), t * 16 + it16],
                zeros, mask=tail_mask | (t < jnp.full((16,), 62, jnp.int32)))
            return carry

        for b in range(_NBUF):
            lax.fori_loop(0, n1 * 63, zrow_body, b)

        def put(j, b):
            c0 = xv[j, 0:16]
            c1 = xv[j, pl.ds(n1 - 16, 16)]
            plsc.store_scatter(buf.at[b], [s0, c0], ones)
            plsc.store_scatter(buf.at[b], [s1, c1], ones)
            pltpu.make_async_copy(buf.at[b], o_hbm.at[r0 + j], sem.at[b]).start()

        def reclaim(j, b):
            pltpu.make_async_copy(buf.at[b], o_hbm.at[r0 + j], sem.at[b]).wait()
            c0 = xv[j, 0:16]
            c1 = xv[j, pl.ds(n1 - 16, 16)]
            plsc.store_scatter(buf.at[b], [s0, c0], zeros)
            plsc.store_scatter(buf.at[b], [s1, c1], zeros)

        for b in range(_NBUF):
            put(b, b)

        def body(g, carry):
            for b in range(_NBUF):
                j = g * _NBUF + b
                reclaim(j - _NBUF, b)
                put(j, b)
            return carry

        lax.fori_loop(1, rows_per_w // _NBUF, body, 0)

        for b in range(_NBUF):
            j = rows_per_w - _NBUF + b
            pltpu.make_async_copy(buf.at[b], o_hbm.at[r0 + j], sem.at[b]).wait()

    return _sc(x)
